# baseline (device time: 54762 ns/iter reference)
import jax
import jax.numpy as jnp
from jax import lax
from jax.experimental import pallas as pl
from jax.experimental.pallas import tpu as pltpu

N_DEV = 16


def kernel(x, w_mat):
    m_per, k = x.shape
    _, n_per = w_mat.shape

    def body(x_ref, w_ref, out_ref, gather_ref, send_sems, recv_sems):
        my = lax.axis_index("i")
        left = lax.rem(my + N_DEV - 1, N_DEV)
        right = lax.rem(my + 1, N_DEV)

        barrier_sem = pltpu.get_barrier_semaphore()
        for nbr in (left, right):
            pl.semaphore_signal(
                barrier_sem, inc=1,
                device_id=(nbr,), device_id_type=pl.DeviceIdType.MESH,
            )
        pl.semaphore_wait(barrier_sem, 2)

        gather_ref[my] = x_ref[...].astype(jnp.bfloat16)

        for h in range(N_DEV - 1):
            slot = lax.rem(my + 2 * N_DEV - h, N_DEV)
            rdma = pltpu.make_async_remote_copy(
                src_ref=gather_ref.at[slot],
                dst_ref=gather_ref.at[slot],
                send_sem=send_sems.at[h],
                recv_sem=recv_sems.at[h],
                device_id=(right,),
                device_id_type=pl.DeviceIdType.MESH,
            )
            rdma.start()
            rdma.wait()

        xg = gather_ref[...].reshape(N_DEV * m_per, k)
        y = jnp.dot(
            xg, w_ref[...].astype(jnp.bfloat16),
            preferred_element_type=jnp.float32,
        )
        out_ref[...] = y * jax.nn.sigmoid(y)

    return pl.pallas_call(
        body,
        out_shape=jax.ShapeDtypeStruct((N_DEV * m_per, n_per), jnp.float32),
        in_specs=[
            pl.BlockSpec(memory_space=pltpu.VMEM),
            pl.BlockSpec(memory_space=pltpu.VMEM),
        ],
        out_specs=pl.BlockSpec(memory_space=pltpu.VMEM),
        scratch_shapes=[
            pltpu.VMEM((N_DEV, m_per, k), jnp.bfloat16),
            pltpu.SemaphoreType.DMA((N_DEV - 1,)),
            pltpu.SemaphoreType.DMA((N_DEV - 1,)),
        ],
        compiler_params=pltpu.CompilerParams(collective_id=0),
    )(x, w_mat)


# device time: 37693 ns/iter; 1.4528x vs baseline; 1.4528x over previous
import jax
import jax.numpy as jnp
from jax import lax
from jax.experimental import pallas as pl
from jax.experimental.pallas import tpu as pltpu

N_DEV = 16
N_R = N_DEV // 2
N_L = N_DEV - 1 - N_R

RING = (0, 4, 8, 12, 15, 11, 7, 3, 2, 6, 10, 14, 13, 9, 5, 1)
POS = tuple(RING.index(p) for p in range(N_DEV))


def kernel(x, w_mat):
    m_per, k = x.shape
    _, n_per = w_mat.shape

    ring_c = jnp.asarray(RING, jnp.int32)
    pos_c = jnp.asarray(POS, jnp.int32)
    my = lax.axis_index("i")
    r = pos_c[my]
    nbrs = jnp.stack([
        ring_c[(r - 1) % N_DEV],
        ring_c[(r + 1) % N_DEV],
    ]).astype(jnp.int32)
    rs_slots = ring_c[(r - jnp.arange(N_R + 1)) % N_DEV].astype(jnp.int32)
    ls_slots = ring_c[(r + jnp.arange(N_L + 1)) % N_DEV].astype(jnp.int32)

    def body(nbr_ref, rs_ref, ls_ref, x_ref, w_ref, out_ref,
             gather_ref, rs_send, rs_recv, ls_send, ls_recv):
        left = nbr_ref[0]
        right = nbr_ref[1]
        my_id = rs_ref[0]

        barrier_sem = pltpu.get_barrier_semaphore()
        for nbr in (left, right):
            pl.semaphore_signal(
                barrier_sem, inc=1,
                device_id=(nbr,), device_id_type=pl.DeviceIdType.MESH,
            )
        pl.semaphore_wait(barrier_sem, 2)

        gather_ref[my_id] = x_ref[...].astype(jnp.bfloat16)

        def send(slot, sem_arr, h, dev):
            rdma = pltpu.make_async_remote_copy(
                src_ref=gather_ref.at[slot],
                dst_ref=gather_ref.at[slot],
                send_sem=sem_arr.at[h],
                recv_sem=(rs_recv if sem_arr is rs_send else ls_recv).at[h],
                device_id=(dev,),
                device_id_type=pl.DeviceIdType.MESH,
            )
            rdma.start()
            return rdma

        def wait_recv(slot, recv_arr, h):
            rdma = pltpu.make_async_remote_copy(
                src_ref=gather_ref.at[slot],
                dst_ref=gather_ref.at[slot],
                send_sem=(rs_send if recv_arr is rs_recv else ls_send).at[h],
                recv_sem=recv_arr.at[h],
                device_id=(right,),
                device_id_type=pl.DeviceIdType.MESH,
            )
            rdma.wait_recv()

        started = []
        started.append(send(rs_ref[0], rs_send, 0, right))
        started.append(send(ls_ref[0], ls_send, 0, left))
        for h in range(1, N_R):
            wait_recv(rs_ref[h], rs_recv, h - 1)
            started.append(send(rs_ref[h], rs_send, h, right))
            if h < N_L:
                wait_recv(ls_ref[h], ls_recv, h - 1)
                started.append(send(ls_ref[h], ls_send, h, left))
        wait_recv(rs_ref[N_R], rs_recv, N_R - 1)
        wait_recv(ls_ref[N_L], ls_recv, N_L - 1)
        for rdma in started:
            rdma.wait_send()

        xg = gather_ref[...].reshape(N_DEV * m_per, k)
        y = jnp.dot(
            xg, w_ref[...].astype(jnp.bfloat16),
            preferred_element_type=jnp.float32,
        )
        out_ref[...] = y * jax.nn.sigmoid(y)

    return pl.pallas_call(
        body,
        out_shape=jax.ShapeDtypeStruct((N_DEV * m_per, n_per), jnp.float32),
        in_specs=[
            pl.BlockSpec(memory_space=pltpu.SMEM),
            pl.BlockSpec(memory_space=pltpu.SMEM),
            pl.BlockSpec(memory_space=pltpu.SMEM),
            pl.BlockSpec(memory_space=pltpu.VMEM),
            pl.BlockSpec(memory_space=pltpu.VMEM),
        ],
        out_specs=pl.BlockSpec(memory_space=pltpu.VMEM),
        scratch_shapes=[
            pltpu.VMEM((N_DEV, m_per, k), jnp.bfloat16),
            pltpu.SemaphoreType.DMA((N_R,)),
            pltpu.SemaphoreType.DMA((N_R,)),
            pltpu.SemaphoreType.DMA((N_L,)),
            pltpu.SemaphoreType.DMA((N_L,)),
        ],
        compiler_params=pltpu.CompilerParams(collective_id=0),
    )(nbrs, rs_slots, ls_slots, x, w_mat)


# device time: 31211 ns/iter; 1.7546x vs baseline; 1.2077x over previous
import jax
import jax.numpy as jnp
from jax import lax
from jax.experimental import pallas as pl
from jax.experimental.pallas import tpu as pltpu

N_DEV = 16
N_R = N_DEV // 2
N_L = N_DEV - 1 - N_R


def _ring(t):
    t = lax.rem(t + 2 * N_DEV, N_DEV)
    q = t // 4
    j = t % 4
    return jnp.where(
        q == 0, 4 * j,
        jnp.where(q == 1, 15 - 4 * j,
                  jnp.where(q == 2, 4 * j + 2, 13 - 4 * j)),
    )


def _pos(p):
    z = p // 4
    c = p % 4
    return jnp.where(
        c == 0, z,
        jnp.where(c == 1, 15 - z, jnp.where(c == 2, 8 + z, 7 - z)),
    )


def kernel(x, w_mat):
    m_per, k = x.shape
    _, n_per = w_mat.shape

    def body(x_ref, w_ref, out_ref,
             gather_ref, rs_send, rs_recv, ls_send, ls_recv):
        my = lax.axis_index("i")
        r = _pos(my)
        left = _ring(r - 1)
        right = _ring(r + 1)

        barrier_sem = pltpu.get_barrier_semaphore()
        for nbr in (left, right):
            pl.semaphore_signal(
                barrier_sem, inc=1,
                device_id=(nbr,), device_id_type=pl.DeviceIdType.MESH,
            )
        pl.semaphore_wait(barrier_sem, 2)

        gather_ref[my] = x_ref[...].astype(jnp.bfloat16)

        def send(slot, send_arr, recv_arr, h, dev):
            rdma = pltpu.make_async_remote_copy(
                src_ref=gather_ref.at[slot],
                dst_ref=gather_ref.at[slot],
                send_sem=send_arr.at[h],
                recv_sem=recv_arr.at[h],
                device_id=(dev,),
                device_id_type=pl.DeviceIdType.MESH,
            )
            rdma.start()
            return rdma

        def wait_recv(slot, send_arr, recv_arr, h):
            rdma = pltpu.make_async_remote_copy(
                src_ref=gather_ref.at[slot],
                dst_ref=gather_ref.at[slot],
                send_sem=send_arr.at[h],
                recv_sem=recv_arr.at[h],
                device_id=(right,),
                device_id_type=pl.DeviceIdType.MESH,
            )
            rdma.wait_recv()

        started = []
        started.append(send(my, rs_send, rs_recv, 0, right))
        started.append(send(my, ls_send, ls_recv, 0, left))
        for h in range(1, N_R):
            wait_recv(_ring(r - h), rs_send, rs_recv, h - 1)
            started.append(send(_ring(r - h), rs_send, rs_recv, h, right))
            if h < N_L:
                wait_recv(_ring(r + h), ls_send, ls_recv, h - 1)
                started.append(send(_ring(r + h), ls_send, ls_recv, h, left))
        wait_recv(_ring(r - N_R), rs_send, rs_recv, N_R - 1)
        wait_recv(_ring(r + N_L), ls_send, ls_recv, N_L - 1)
        for rdma in started:
            rdma.wait_send()

        xg = gather_ref[...].reshape(N_DEV * m_per, k)
        y = jnp.dot(
            xg, w_ref[...].astype(jnp.bfloat16),
            preferred_element_type=jnp.float32,
        )
        out_ref[...] = y * jax.nn.sigmoid(y)

    return pl.pallas_call(
        body,
        out_shape=jax.ShapeDtypeStruct((N_DEV * m_per, n_per), jnp.float32),
        in_specs=[
            pl.BlockSpec(memory_space=pltpu.VMEM),
            pl.BlockSpec(memory_space=pltpu.VMEM),
        ],
        out_specs=pl.BlockSpec(memory_space=pltpu.VMEM),
        scratch_shapes=[
            pltpu.VMEM((N_DEV, m_per, k), jnp.bfloat16),
            pltpu.SemaphoreType.DMA((N_R,)),
            pltpu.SemaphoreType.DMA((N_R,)),
            pltpu.SemaphoreType.DMA((N_L,)),
            pltpu.SemaphoreType.DMA((N_L,)),
        ],
        compiler_params=pltpu.CompilerParams(collective_id=0),
    )(x, w_mat)


# device time: 27053 ns/iter; 2.0242x vs baseline; 1.1537x over previous
import jax
import jax.numpy as jnp
from jax import lax
from jax.experimental import pallas as pl
from jax.experimental.pallas import tpu as pltpu

N_DEV = 16
N_R = N_DEV // 2
N_L = N_DEV - 1 - N_R
N_SEG = 2


def _ring(t):
    t = lax.rem(t + 2 * N_DEV, N_DEV)
    q = t // 4
    j = t % 4
    return jnp.where(
        q == 0, 4 * j,
        jnp.where(q == 1, 15 - 4 * j,
                  jnp.where(q == 2, 4 * j + 2, 13 - 4 * j)),
    )


def _pos(p):
    z = p // 4
    c = p % 4
    return jnp.where(
        c == 0, z,
        jnp.where(c == 1, 15 - z, jnp.where(c == 2, 8 + z, 7 - z)),
    )


def kernel(x, w_mat):
    m_per, k = x.shape
    _, n_per = w_mat.shape

    def body(x_ref, w_ref, out_ref,
             gather_ref, rs_send, rs_recv, ls_send, ls_recv):
        my = lax.axis_index("i")
        r = _pos(my)
        left = _ring(r - 1)
        right = _ring(r + 1)

        barrier_sem = pltpu.get_barrier_semaphore()
        for nbr in (left, right):
            pl.semaphore_signal(
                barrier_sem, inc=1,
                device_id=(nbr,), device_id_type=pl.DeviceIdType.MESH,
            )
        pl.semaphore_wait(barrier_sem, 2)

        gather_ref[my] = x_ref[...].astype(jnp.bfloat16)

        m_seg = m_per // N_SEG

        def send(slot, seg, send_arr, recv_arr, h, dev):
            rdma = pltpu.make_async_remote_copy(
                src_ref=gather_ref.at[slot, pl.ds(seg * m_seg, m_seg)],
                dst_ref=gather_ref.at[slot, pl.ds(seg * m_seg, m_seg)],
                send_sem=send_arr.at[h, seg],
                recv_sem=recv_arr.at[h, seg],
                device_id=(dev,),
                device_id_type=pl.DeviceIdType.MESH,
            )
            rdma.start()
            return rdma

        def wait_recv(slot, seg, send_arr, recv_arr, h):
            rdma = pltpu.make_async_remote_copy(
                src_ref=gather_ref.at[slot, pl.ds(seg * m_seg, m_seg)],
                dst_ref=gather_ref.at[slot, pl.ds(seg * m_seg, m_seg)],
                send_sem=send_arr.at[h, seg],
                recv_sem=recv_arr.at[h, seg],
                device_id=(right,),
                device_id_type=pl.DeviceIdType.MESH,
            )
            rdma.wait_recv()

        started = []
        for s in range(N_SEG):
            started.append(send(my, s, rs_send, rs_recv, 0, right))
            started.append(send(my, s, ls_send, ls_recv, 0, left))
        for h in range(1, N_R):
            for s in range(N_SEG):
                wait_recv(_ring(r - h), s, rs_send, rs_recv, h - 1)
                started.append(send(_ring(r - h), s, rs_send, rs_recv, h, right))
                if h < N_L:
                    wait_recv(_ring(r + h), s, ls_send, ls_recv, h - 1)
                    started.append(send(_ring(r + h), s, ls_send, ls_recv, h, left))
        for s in range(N_SEG):
            wait_recv(_ring(r - N_R), s, rs_send, rs_recv, N_R - 1)
            wait_recv(_ring(r + N_L), s, ls_send, ls_recv, N_L - 1)
        for rdma in started:
            rdma.wait_send()

        xg = gather_ref[...].reshape(N_DEV * m_per, k)
        y = jnp.dot(
            xg, w_ref[...].astype(jnp.bfloat16),
            preferred_element_type=jnp.float32,
        )
        out_ref[...] = y * jax.nn.sigmoid(y)

    return pl.pallas_call(
        body,
        out_shape=jax.ShapeDtypeStruct((N_DEV * m_per, n_per), jnp.float32),
        in_specs=[
            pl.BlockSpec(memory_space=pltpu.VMEM),
            pl.BlockSpec(memory_space=pltpu.VMEM),
        ],
        out_specs=pl.BlockSpec(memory_space=pltpu.VMEM),
        scratch_shapes=[
            pltpu.VMEM((N_DEV, m_per, k), jnp.bfloat16),
            pltpu.SemaphoreType.DMA((N_R, N_SEG)),
            pltpu.SemaphoreType.DMA((N_R, N_SEG)),
            pltpu.SemaphoreType.DMA((N_L, N_SEG)),
            pltpu.SemaphoreType.DMA((N_L, N_SEG)),
        ],
        compiler_params=pltpu.CompilerParams(collective_id=0),
    )(x, w_mat)


# device time: 25582 ns/iter; 2.1406x vs baseline; 1.0575x over previous
import jax
import jax.numpy as jnp
from jax import lax
from jax.experimental import pallas as pl
from jax.experimental.pallas import tpu as pltpu

N_DEV = 16
N_R = N_DEV // 2
N_L = N_DEV - 1 - N_R
N_SEG = 4


def _ring(t):
    t = lax.rem(t + 2 * N_DEV, N_DEV)
    q = t // 4
    j = t % 4
    return jnp.where(
        q == 0, 4 * j,
        jnp.where(q == 1, 15 - 4 * j,
                  jnp.where(q == 2, 4 * j + 2, 13 - 4 * j)),
    )


def _pos(p):
    z = p // 4
    c = p % 4
    return jnp.where(
        c == 0, z,
        jnp.where(c == 1, 15 - z, jnp.where(c == 2, 8 + z, 7 - z)),
    )


def kernel(x, w_mat):
    m_per, k = x.shape
    _, n_per = w_mat.shape

    def body(x_ref, w_ref, out_ref,
             gather_ref, rs_send, rs_recv, ls_send, ls_recv):
        my = lax.axis_index("i")
        r = _pos(my)
        left = _ring(r - 1)
        right = _ring(r + 1)

        barrier_sem = pltpu.get_barrier_semaphore()
        for nbr in (left, right):
            pl.semaphore_signal(
                barrier_sem, inc=1,
                device_id=(nbr,), device_id_type=pl.DeviceIdType.MESH,
            )
        pl.semaphore_wait(barrier_sem, 2)

        gather_ref[my] = x_ref[...].astype(jnp.bfloat16)

        m_seg = m_per // N_SEG

        def send(slot, seg, send_arr, recv_arr, h, dev):
            rdma = pltpu.make_async_remote_copy(
                src_ref=gather_ref.at[slot, pl.ds(seg * m_seg, m_seg)],
                dst_ref=gather_ref.at[slot, pl.ds(seg * m_seg, m_seg)],
                send_sem=send_arr.at[h, seg],
                recv_sem=recv_arr.at[h, seg],
                device_id=(dev,),
                device_id_type=pl.DeviceIdType.MESH,
            )
            rdma.start()
            return rdma

        def wait_recv(slot, seg, send_arr, recv_arr, h):
            rdma = pltpu.make_async_remote_copy(
                src_ref=gather_ref.at[slot, pl.ds(seg * m_seg, m_seg)],
                dst_ref=gather_ref.at[slot, pl.ds(seg * m_seg, m_seg)],
                send_sem=send_arr.at[h, seg],
                recv_sem=recv_arr.at[h, seg],
                device_id=(right,),
                device_id_type=pl.DeviceIdType.MESH,
            )
            rdma.wait_recv()

        started = []
        for s in range(N_SEG):
            started.append(send(my, s, rs_send, rs_recv, 0, right))
            started.append(send(my, s, ls_send, ls_recv, 0, left))
        for h in range(1, N_R):
            for s in range(N_SEG):
                wait_recv(_ring(r - h), s, rs_send, rs_recv, h - 1)
                started.append(send(_ring(r - h), s, rs_send, rs_recv, h, right))
                if h < N_L:
                    wait_recv(_ring(r + h), s, ls_send, ls_recv, h - 1)
                    started.append(send(_ring(r + h), s, ls_send, ls_recv, h, left))
        for s in range(N_SEG):
            wait_recv(_ring(r - N_R), s, rs_send, rs_recv, N_R - 1)
            wait_recv(_ring(r + N_L), s, ls_send, ls_recv, N_L - 1)
        for rdma in started:
            rdma.wait_send()

        xg = gather_ref[...].reshape(N_DEV * m_per, k)
        y = jnp.dot(
            xg, w_ref[...].astype(jnp.bfloat16),
            preferred_element_type=jnp.float32,
        )
        out_ref[...] = y * jax.nn.sigmoid(y)

    return pl.pallas_call(
        body,
        out_shape=jax.ShapeDtypeStruct((N_DEV * m_per, n_per), jnp.float32),
        in_specs=[
            pl.BlockSpec(memory_space=pltpu.VMEM),
            pl.BlockSpec(memory_space=pltpu.VMEM),
        ],
        out_specs=pl.BlockSpec(memory_space=pltpu.VMEM),
        scratch_shapes=[
            pltpu.VMEM((N_DEV, m_per, k), jnp.bfloat16),
            pltpu.SemaphoreType.DMA((N_R, N_SEG)),
            pltpu.SemaphoreType.DMA((N_R, N_SEG)),
            pltpu.SemaphoreType.DMA((N_L, N_SEG)),
            pltpu.SemaphoreType.DMA((N_L, N_SEG)),
        ],
        compiler_params=pltpu.CompilerParams(collective_id=0),
    )(x, w_mat)


# device time: 25540 ns/iter; 2.1442x vs baseline; 1.0016x over previous
import jax
import jax.numpy as jnp
from jax import lax
from jax.experimental import pallas as pl
from jax.experimental.pallas import tpu as pltpu

N_DEV = 16
N_R = N_DEV // 2
N_L = N_DEV - 1 - N_R
N_SEG = 4


def _ring(t):
    t = lax.rem(t + 2 * N_DEV, N_DEV)
    q = t // 4
    j = t % 4
    return jnp.where(
        q == 0, 4 * j,
        jnp.where(q == 1, 15 - 4 * j,
                  jnp.where(q == 2, 4 * j + 2, 13 - 4 * j)),
    )


def _pos(p):
    z = p // 4
    c = p % 4
    return jnp.where(
        c == 0, z,
        jnp.where(c == 1, 15 - z, jnp.where(c == 2, 8 + z, 7 - z)),
    )


def kernel(x, w_mat):
    m_per, k = x.shape
    _, n_per = w_mat.shape
    m_seg = m_per // N_SEG

    def body(x_ref, w_ref, out_ref,
             gather_ref, rs_send, rs_recv, ls_send, ls_recv):
        my = lax.axis_index("i")
        r = _pos(my)
        left = _ring(r - 1)
        right = _ring(r + 1)

        barrier_sem = pltpu.get_barrier_semaphore()
        for nbr in (left, right):
            pl.semaphore_signal(
                barrier_sem, inc=1,
                device_id=(nbr,), device_id_type=pl.DeviceIdType.MESH,
            )
        pl.semaphore_wait(barrier_sem, 2)

        gather_ref[0] = x_ref[...].astype(jnp.bfloat16)

        def mk(src_slot, dst_slot, seg, send_arr, recv_arr, h, dev):
            return pltpu.make_async_remote_copy(
                src_ref=gather_ref.at[src_slot, pl.ds(seg * m_seg, m_seg)],
                dst_ref=gather_ref.at[dst_slot, pl.ds(seg * m_seg, m_seg)],
                send_sem=send_arr.at[h, seg],
                recv_sem=recv_arr.at[h, seg],
                device_id=(dev,),
                device_id_type=pl.DeviceIdType.MESH,
            )

        started = []
        for s in range(N_SEG):
            d = mk(0, 1, s, rs_send, rs_recv, 0, right)
            d.start()
            started.append(d)
            d = mk(0, 9, s, ls_send, ls_recv, 0, left)
            d.start()
            started.append(d)
        for h in range(1, N_R):
            for s in range(N_SEG):
                mk(h, h, s, rs_send, rs_recv, h - 1, right).wait_recv()
                d = mk(h, h + 1, s, rs_send, rs_recv, h, right)
                d.start()
                started.append(d)
                if h < N_L:
                    mk(8 + h, 8 + h, s, ls_send, ls_recv, h - 1, left).wait_recv()
                    d = mk(8 + h, 9 + h, s, ls_send, ls_recv, h, left)
                    d.start()
                    started.append(d)
        for s in range(N_SEG):
            mk(8, 8, s, rs_send, rs_recv, N_R - 1, right).wait_recv()
            mk(15, 15, s, ls_send, ls_recv, N_L - 1, left).wait_recv()
        for d in started:
            d.wait_send()

        xg = gather_ref[...].reshape(N_DEV * m_per, k)
        y = jnp.dot(
            xg, w_ref[...].astype(jnp.bfloat16),
            preferred_element_type=jnp.float32,
        )
        y = y * jax.nn.sigmoid(y)
        out_ref[pl.ds(my * m_per, m_per), :] = y[0:m_per, :]
        for j in range(1, N_DEV):
            origin = _ring(r - j) if j <= N_R else _ring(r + (j - 8))
            out_ref[pl.ds(origin * m_per, m_per), :] = (
                y[j * m_per:(j + 1) * m_per, :]
            )

    return pl.pallas_call(
        body,
        out_shape=jax.ShapeDtypeStruct((N_DEV * m_per, n_per), jnp.float32),
        in_specs=[
            pl.BlockSpec(memory_space=pltpu.VMEM),
            pl.BlockSpec(memory_space=pltpu.VMEM),
        ],
        out_specs=pl.BlockSpec(memory_space=pltpu.VMEM),
        scratch_shapes=[
            pltpu.VMEM((N_DEV, m_per, k), jnp.bfloat16),
            pltpu.SemaphoreType.DMA((N_R, N_SEG)),
            pltpu.SemaphoreType.DMA((N_R, N_SEG)),
            pltpu.SemaphoreType.DMA((N_L, N_SEG)),
            pltpu.SemaphoreType.DMA((N_L, N_SEG)),
        ],
        compiler_params=pltpu.CompilerParams(collective_id=0),
    )(x, w_mat)


# device time: 20731 ns/iter; 2.6416x vs baseline; 1.2320x over previous
import jax
import jax.numpy as jnp
from jax import lax
from jax.experimental import pallas as pl
from jax.experimental.pallas import tpu as pltpu

N_DEV = 16
N_SEG = 4


def _ring(t):
    t = lax.rem(t + 2 * N_DEV, N_DEV)
    q = t // 4
    j = t % 4
    return jnp.where(
        q == 0, 4 * j,
        jnp.where(q == 1, 15 - 4 * j,
                  jnp.where(q == 2, 4 * j + 2, 13 - 4 * j)),
    )


def _pos(p):
    z = p // 4
    c = p % 4
    return jnp.where(
        c == 0, z,
        jnp.where(c == 1, 15 - z, jnp.where(c == 2, 8 + z, 7 - z)),
    )


_SCHEDULE = (
    ((None, ((0, 1, "R"), (0, 9, "L"), (0, 8, "C"))),),
    ((1, ((1, 2, "R"),)), (9, ((9, 10, "L"),))),
    ((8, ((8, 7, "L"), (8, 15, "R"))),),
    ((2, ((2, 3, "R"),)), (10, ((10, 11, "L"),))),
    ((7, ((7, 6, "L"),)), (15, ((15, 14, "R"),))),
    ((3, ((3, 4, "R"),)), (11, ((11, 12, "L"),))),
    ((6, ((6, 5, "L"),)), (14, ((14, 13, "R"),))),
    ((4, ()), (12, ()), (5, ()), (13, ())),
)


def kernel(x, w_mat):
    m_per, k = x.shape
    _, n_per = w_mat.shape
    m_seg = m_per // N_SEG

    def body(x_ref, w_ref, out_ref, gather_ref, send_sems, recv_sems):
        my = lax.axis_index("i")
        r = _pos(my)
        left = _ring(r - 1)
        right = _ring(r + 1)
        chord = _ring(r + 8)

        barrier_sem = pltpu.get_barrier_semaphore()
        for nbr in (left, right, chord):
            pl.semaphore_signal(
                barrier_sem, inc=1,
                device_id=(nbr,), device_id_type=pl.DeviceIdType.MESH,
            )
        pl.semaphore_wait(barrier_sem, 3)

        gather_ref[0] = x_ref[...].astype(jnp.bfloat16)

        dev_of = {"R": right, "L": left, "C": chord}

        def mk(src_slot, dst_slot, seg, send_idx, dev):
            return pltpu.make_async_remote_copy(
                src_ref=gather_ref.at[src_slot, pl.ds(seg * m_seg, m_seg)],
                dst_ref=gather_ref.at[dst_slot, pl.ds(seg * m_seg, m_seg)],
                send_sem=send_sems.at[send_idx, seg],
                recv_sem=recv_sems.at[dst_slot, seg],
                device_id=(dev,),
                device_id_type=pl.DeviceIdType.MESH,
            )

        started = []
        send_idx = 0
        for stage in _SCHEDULE:
            base = send_idx
            for s in range(N_SEG):
                idx = base
                for wait_slot, sends in stage:
                    if wait_slot is not None:
                        mk(wait_slot, wait_slot, s, 0, right).wait_recv()
                    for si, (src, dst, dv) in enumerate(sends):
                        d = mk(src, dst, s, idx + si, dev_of[dv])
                        d.start()
                        started.append(d)
                    idx += len(sends)
            send_idx = idx
        for d in started:
            d.wait_send()

        xg = gather_ref[...].reshape(N_DEV * m_per, k)
        y = jnp.dot(
            xg, w_ref[...].astype(jnp.bfloat16),
            preferred_element_type=jnp.float32,
        )
        y = y * jax.nn.sigmoid(y)
        out_ref[pl.ds(my * m_per, m_per), :] = y[0:m_per, :]
        for j in range(1, N_DEV):
            origin = _ring(r - j) if j <= 8 else _ring(r + (j - 8))
            out_ref[pl.ds(origin * m_per, m_per), :] = (
                y[j * m_per:(j + 1) * m_per, :]
            )

    n_sends = sum(len(s) for stage in _SCHEDULE for _, s in stage)
    return pl.pallas_call(
        body,
        out_shape=jax.ShapeDtypeStruct((N_DEV * m_per, n_per), jnp.float32),
        in_specs=[
            pl.BlockSpec(memory_space=pltpu.VMEM),
            pl.BlockSpec(memory_space=pltpu.VMEM),
        ],
        out_specs=pl.BlockSpec(memory_space=pltpu.VMEM),
        scratch_shapes=[
            pltpu.VMEM((N_DEV, m_per, k), jnp.bfloat16),
            pltpu.SemaphoreType.DMA((n_sends, N_SEG)),
            pltpu.SemaphoreType.DMA((N_DEV, N_SEG)),
        ],
        compiler_params=pltpu.CompilerParams(collective_id=0),
    )(x, w_mat)


# device time: 20300 ns/iter; 2.6976x vs baseline; 1.0212x over previous
import jax
import jax.numpy as jnp
from jax import lax
from jax.experimental import pallas as pl
from jax.experimental.pallas import tpu as pltpu

N_DEV = 16
N_SEG = 4


def _ring(t):
    t = lax.rem(t + 2 * N_DEV, N_DEV)
    q = t // 4
    j = t % 4
    return jnp.where(
        q == 0, 4 * j,
        jnp.where(q == 1, 15 - 4 * j,
                  jnp.where(q == 2, 4 * j + 2, 13 - 4 * j)),
    )


def _pos(p):
    z = p // 4
    c = p % 4
    return jnp.where(
        c == 0, z,
        jnp.where(c == 1, 15 - z, jnp.where(c == 2, 8 + z, 7 - z)),
    )


_SCHEDULE = (
    ((None, ((0, 1, "R"), (0, 9, "L"), (0, 8, "C"))),),
    ((1, ((1, 2, "R"),)), (9, ((9, 10, "L"),))),
    ((8, ((8, 7, "L"), (8, 15, "R"))),),
    ((2, ((2, 3, "R"),)), (10, ((10, 11, "L"),))),
    ((7, ((7, 6, "L"),)), (15, ((15, 14, "R"),))),
    ((3, ((3, 4, "R"),)), (11, ((11, 12, "L"),))),
    ((6, ((6, 5, "L"),)), (14, ((14, 13, "R"),))),
    ((4, ()), (12, ()), (5, ()), (13, ())),
)


def kernel(x, w_mat):
    m_per, k = x.shape
    _, n_per = w_mat.shape
    m_seg = m_per // N_SEG

    def body(x_ref, w_ref, out_ref, gather_ref, w_bf, send_sems, recv_sems):
        my = lax.axis_index("i")
        r = _pos(my)
        left = _ring(r - 1)
        right = _ring(r + 1)
        chord = _ring(r + 8)

        gather_ref[0] = x_ref[...].astype(jnp.bfloat16)
        w_bf[...] = w_ref[...].astype(jnp.bfloat16)

        barrier_sem = pltpu.get_barrier_semaphore()
        for nbr in (left, right, chord):
            pl.semaphore_signal(
                barrier_sem, inc=1,
                device_id=(nbr,), device_id_type=pl.DeviceIdType.MESH,
            )
        pl.semaphore_wait(barrier_sem, 3)

        dev_of = {"R": right, "L": left, "C": chord}

        def mk(src_slot, dst_slot, seg, send_idx, dev):
            return pltpu.make_async_remote_copy(
                src_ref=gather_ref.at[src_slot, pl.ds(seg * m_seg, m_seg)],
                dst_ref=gather_ref.at[dst_slot, pl.ds(seg * m_seg, m_seg)],
                send_sem=send_sems.at[send_idx, seg],
                recv_sem=recv_sems.at[dst_slot, seg],
                device_id=(dev,),
                device_id_type=pl.DeviceIdType.MESH,
            )

        def compute_block(j):
            if j == 0:
                origin = my
            elif j <= 8:
                origin = _ring(r - j)
            else:
                origin = _ring(r + (j - 8))
            y = jnp.dot(
                gather_ref[j], w_bf[...],
                preferred_element_type=jnp.float32,
            )
            out_ref[pl.ds(origin * m_per, m_per), :] = y * jax.nn.sigmoid(y)

        started = []
        send_idx = 0
        for stage in _SCHEDULE:
            base = send_idx
            for s in range(N_SEG):
                idx = base
                for wait_slot, sends in stage:
                    if wait_slot is not None:
                        mk(wait_slot, wait_slot, s, 0, right).wait_recv()
                    for si, (src, dst, dv) in enumerate(sends):
                        d = mk(src, dst, s, idx + si, dev_of[dv])
                        d.start()
                        started.append(d)
                    idx += len(sends)
            send_idx = idx
            for wait_slot, _ in stage:
                compute_block(0 if wait_slot is None else wait_slot)
        for d in started:
            d.wait_send()

    n_sends = sum(len(s) for stage in _SCHEDULE for _, s in stage)
    return pl.pallas_call(
        body,
        out_shape=jax.ShapeDtypeStruct((N_DEV * m_per, n_per), jnp.float32),
        in_specs=[
            pl.BlockSpec(memory_space=pltpu.VMEM),
            pl.BlockSpec(memory_space=pltpu.VMEM),
        ],
        out_specs=pl.BlockSpec(memory_space=pltpu.VMEM),
        scratch_shapes=[
            pltpu.VMEM((N_DEV, m_per, k), jnp.bfloat16),
            pltpu.VMEM((k, n_per), jnp.bfloat16),
            pltpu.SemaphoreType.DMA((n_sends, N_SEG)),
            pltpu.SemaphoreType.DMA((N_DEV, N_SEG)),
        ],
        compiler_params=pltpu.CompilerParams(collective_id=0),
    )(x, w_mat)
